# baseline (speedup 1.0000x reference)
import functools

import jax
import jax.numpy as jnp
from jax import lax
from jax.experimental import pallas as pl
from jax.experimental.pallas import tpu as pltpu
from jax.experimental.pallas import tpu_sc as plsc

KNN = 16
TQ = 256
TQ2 = 256
SC_CHUNK = 128
TW = 256


def stage_a_body(coords_ref, coordsq_ref, featsT_ref, WinT_ref, Wt1T_ref,
                 Wt2T_ref, Wg1T_ref, Wg2T_ref, table_ref, wphi_ref, idx_ref):
    b = pl.program_id(0)
    n_all = coords_ref.shape[-1]
    fT = featsT_ref[0]
    linT = jnp.dot(fT, WinT_ref[...], preferred_element_type=jnp.float32)
    phiT = linT[:, 0:64]
    psiT = linT[:, 64:128]
    alphaT = linT[:, 128:192]
    WgT = jnp.dot(Wg1T_ref[...], Wg2T_ref[...],
                  preferred_element_type=jnp.float32)
    wphiT = jnp.dot(phiT, WgT, preferred_element_type=jnp.float32)
    wpsiT = jnp.dot(psiT, WgT, preferred_element_type=jnp.float32)
    q = coordsq_ref[0]
    t1 = (q[:, 0:1] * Wt1T_ref[0:1, :] + q[:, 1:2] * Wt1T_ref[1:2, :]
          + q[:, 2:3] * Wt1T_ref[2:3, :])
    tcT = jnp.dot(t1, Wt2T_ref[...], preferred_element_type=jnp.float32)
    table_ref[0] = jnp.concatenate(
        [tcT, wpsiT, alphaT, jnp.zeros_like(tcT)], axis=1)
    wphi_ref[0] = wphiT
    a8 = coords_ref[0]
    inner = jnp.dot(q.astype(jnp.bfloat16), a8.astype(jnp.bfloat16),
                    preferred_element_type=jnp.float32)
    q2 = q[:, 0:1] * q[:, 0:1] + q[:, 1:2] * q[:, 1:2] + q[:, 2:3] * q[:, 2:3]
    a2 = (a8[0:1, :] * a8[0:1, :] + a8[1:2, :] * a8[1:2, :]
          + a8[2:3, :] * a8[2:3, :])
    d = (q2 - 2.0 * inner) + a2
    iota = lax.broadcasted_iota(jnp.int32, d.shape, 1)
    inf = jnp.float32(jnp.inf)
    cols = []
    for _ in range(KNN):
        ij = jnp.argmin(d, axis=1, keepdims=True).astype(jnp.int32)
        cols.append(ij)
        d = jnp.where(iota == ij, inf, d)
    idxs = jnp.concatenate(cols, axis=1)
    idx_ref[0] = idxs + b * n_all


def sc_gather_body(table_hbm, idx_hbm, out_hbm, idx_v, buf, sem):
    c = lax.axis_index("c")
    s = lax.axis_index("s")
    wid = s * 2 + c
    rows_total = idx_hbm.shape[0]
    rw = rows_total // 32
    nch = rw // SC_CHUNK

    def body(i, carry):
        off = wid * rw + i * SC_CHUNK
        pltpu.sync_copy(idx_hbm.at[pl.ds(off, SC_CHUNK)], idx_v)
        pltpu.async_copy(table_hbm.at[idx_v], buf, sem).wait()
        pltpu.sync_copy(buf, out_hbm.at[pl.ds(off, SC_CHUNK)])
        return carry

    lax.fori_loop(0, nch, body, 0)


def stage_c_body(G_ref, table_ref, wphi_ref, Wg1T_ref, Wg2T_ref, out_ref):
    WgT = jnp.dot(Wg1T_ref[...], Wg2T_ref[...],
                  preferred_element_type=jnp.float32)
    tcn = table_ref[0][:, 0:64]
    wphin = wphi_ref[0]
    G = G_ref[0]

    def unpack(words):
        wi = lax.bitcast_convert_type(words, jnp.int32)
        lo = lax.bitcast_convert_type(wi << 16, jnp.float32)
        hi = lax.bitcast_convert_type(
            wi & jnp.int32(-65536), jnp.float32)
        return jnp.concatenate([lo, hi], axis=-1)

    tck = G[:, :, 0:64]
    wpsik = unpack(G[:, :, 64:96])
    alphak = unpack(G[:, :, 96:128])
    tcr = jnp.broadcast_to(tcn[:, None, :], (TQ2, KNN, 64))
    wpr = jnp.broadcast_to(wphin[:, None, :], (TQ2, KNN, 64))
    delta = jnp.maximum(tcr - tck, 0.0)
    gam = jnp.dot(delta.reshape(TQ2 * KNN, 64), WgT,
                  preferred_element_type=jnp.float32).reshape(TQ2, KNN, 64)
    gam = jnp.maximum(wpr - wpsik.astype(jnp.float32) + gam, 0.0)
    m = jnp.max(gam, axis=1, keepdims=True)
    e = jnp.exp(gam - m)
    ssum = jnp.sum(e, axis=1)
    acc = jnp.sum(e * (alphak.astype(jnp.float32) + delta), axis=1)
    out_ref[0] = acc / ssum


def kernel(features, coords, W_in, Wt1, Wt2, Wg1, Wg2):
    B, C_in, N = features.shape
    C_out = Wt1.shape[0]
    featsT = jnp.transpose(features, (0, 2, 1))
    coordsq = jnp.transpose(coords, (0, 2, 1))
    coordsq = jnp.pad(coordsq, ((0, 0), (0, 0), (0, 5)))
    coords8 = jnp.pad(coords, ((0, 0), (0, 5), (0, 0)))
    WinT = W_in.T
    Wt1T = Wt1.T
    Wt2T = Wt2.T
    Wg1T = Wg1.T
    Wg2T = Wg2.T

    grid_a = (B, N // TQ)
    table, wphi, idx = pl.pallas_call(
        stage_a_body,
        grid=grid_a,
        in_specs=[
            pl.BlockSpec((1, 8, N), lambda b, qi: (b, 0, 0)),
            pl.BlockSpec((1, TQ, 8), lambda b, qi: (b, qi, 0)),
            pl.BlockSpec((1, TQ, C_in), lambda b, qi: (b, qi, 0)),
            pl.BlockSpec((C_in, 3 * C_out), lambda b, qi: (0, 0)),
            pl.BlockSpec((3, C_out), lambda b, qi: (0, 0)),
            pl.BlockSpec((C_out, C_out), lambda b, qi: (0, 0)),
            pl.BlockSpec((C_out, C_out), lambda b, qi: (0, 0)),
            pl.BlockSpec((C_out, C_out), lambda b, qi: (0, 0)),
        ],
        out_specs=[
            pl.BlockSpec((1, TQ, TW), lambda b, qi: (b, qi, 0)),
            pl.BlockSpec((1, TQ, C_out), lambda b, qi: (b, qi, 0)),
            pl.BlockSpec((1, TQ, KNN), lambda b, qi: (b, qi, 0)),
        ],
        out_shape=[
            jax.ShapeDtypeStruct((B, N, TW), jnp.float32),
            jax.ShapeDtypeStruct((B, N, C_out), jnp.float32),
            jax.ShapeDtypeStruct((B, N, KNN), jnp.int32),
        ],
    )(coords8, coordsq, featsT, WinT, Wt1T, Wt2T, Wg1T, Wg2T)

    R = B * N * KNN
    t2 = table.reshape(B * N, TW)

    def pack(cols):
        cb = cols.astype(jnp.bfloat16)
        pairs = jnp.stack([cb[:, 0:32], cb[:, 32:64]], axis=-1)
        return lax.bitcast_convert_type(pairs, jnp.float32)

    tablep = jnp.concatenate(
        [t2[:, 0:64], pack(t2[:, 64:128]), pack(t2[:, 128:192])], axis=1)
    gather = pl.kernel(
        sc_gather_body,
        out_type=jax.ShapeDtypeStruct((R, 128), jnp.float32),
        mesh=plsc.VectorSubcoreMesh(core_axis_name="c", subcore_axis_name="s"),
        scratch_types=[
            pltpu.VMEM((SC_CHUNK,), jnp.int32),
            pltpu.VMEM((SC_CHUNK, 128), jnp.float32),
            pltpu.SemaphoreType.DMA,
        ],
    )
    G = gather(tablep, idx.reshape(R))

    grid_c = (B, N // TQ2)
    outT = pl.pallas_call(
        stage_c_body,
        grid=grid_c,
        in_specs=[
            pl.BlockSpec((1, TQ2, KNN, 128), lambda b, qi: (b, qi, 0, 0),),
            pl.BlockSpec((1, TQ2, TW), lambda b, qi: (b, qi, 0)),
            pl.BlockSpec((1, TQ2, C_out), lambda b, qi: (b, qi, 0)),
            pl.BlockSpec((C_out, C_out), lambda b, qi: (0, 0)),
            pl.BlockSpec((C_out, C_out), lambda b, qi: (0, 0)),
        ],
        out_specs=pl.BlockSpec((1, TQ2, C_out), lambda b, qi: (b, qi, 0)),
        out_shape=jax.ShapeDtypeStruct((B, N, C_out), jnp.float32),
    )(G.reshape(B, N, KNN, 128), table, wphi, Wg1T, Wg2T)

    return jnp.transpose(outT, (0, 2, 1))


# speedup vs baseline: 1.0469x; 1.0469x over previous
import functools

import jax
import jax.numpy as jnp
from jax import lax
from jax.experimental import pallas as pl
from jax.experimental.pallas import tpu as pltpu
from jax.experimental.pallas import tpu_sc as plsc

KNN = 16
TQ = 256
TQ2 = 256
SC_CHUNK = 128
TW = 256


def stage_a_body(coords_ref, coordsq_ref, featsT_ref, WinT_ref, Wt1T_ref,
                 Wt2T_ref, Wg1T_ref, Wg2T_ref, table_ref, wphi_ref, idx_ref):
    b = pl.program_id(0)
    n_all = coords_ref.shape[-1]
    fT = featsT_ref[0]
    linT = jnp.dot(fT, WinT_ref[...], preferred_element_type=jnp.float32)
    phiT = linT[:, 0:64]
    psiT = linT[:, 64:128]
    alphaT = linT[:, 128:192]
    WgT = jnp.dot(Wg1T_ref[...], Wg2T_ref[...],
                  preferred_element_type=jnp.float32)
    wphiT = jnp.dot(phiT, WgT, preferred_element_type=jnp.float32)
    wpsiT = jnp.dot(psiT, WgT, preferred_element_type=jnp.float32)
    q = coordsq_ref[0]
    t1 = (q[:, 0:1] * Wt1T_ref[0:1, :] + q[:, 1:2] * Wt1T_ref[1:2, :]
          + q[:, 2:3] * Wt1T_ref[2:3, :])
    tcT = jnp.dot(t1, Wt2T_ref[...], preferred_element_type=jnp.float32)

    def pack(cols):
        r = cols.astype(jnp.bfloat16).astype(jnp.float32)
        lo = lax.shift_right_logical(
            lax.bitcast_convert_type(r[:, 0:32], jnp.int32), 16)
        hi = lax.bitcast_convert_type(r[:, 32:64], jnp.int32)
        return lax.bitcast_convert_type(hi | lo, jnp.float32)

    table_ref[0] = jnp.concatenate([tcT, pack(wpsiT), pack(alphaT)], axis=1)
    wphi_ref[0] = wphiT
    a8 = coords_ref[0]
    inner = jnp.dot(q.astype(jnp.bfloat16), a8.astype(jnp.bfloat16),
                    preferred_element_type=jnp.float32)
    q2 = q[:, 0:1] * q[:, 0:1] + q[:, 1:2] * q[:, 1:2] + q[:, 2:3] * q[:, 2:3]
    a2 = (a8[0:1, :] * a8[0:1, :] + a8[1:2, :] * a8[1:2, :]
          + a8[2:3, :] * a8[2:3, :])
    d = (q2 - 2.0 * inner) + a2
    iota = lax.broadcasted_iota(jnp.int32, d.shape, 1)
    inf = jnp.float32(jnp.inf)
    cols = []
    for _ in range(KNN):
        ij = jnp.argmin(d, axis=1, keepdims=True).astype(jnp.int32)
        cols.append(ij)
        d = jnp.where(iota == ij, inf, d)
    idxs = jnp.concatenate(cols, axis=1)
    idx_ref[0] = idxs + b * n_all


def sc_gather_body(table_hbm, idx_hbm, out_hbm, idx_v, buf, sem):
    c = lax.axis_index("c")
    s = lax.axis_index("s")
    wid = s * 2 + c
    rows_total = idx_hbm.shape[0]
    rw = rows_total // 32
    nch = rw // SC_CHUNK

    def body(i, carry):
        off = wid * rw + i * SC_CHUNK
        pltpu.sync_copy(idx_hbm.at[pl.ds(off, SC_CHUNK)], idx_v)
        pltpu.async_copy(table_hbm.at[idx_v], buf, sem).wait()
        pltpu.sync_copy(buf, out_hbm.at[pl.ds(off, SC_CHUNK)])
        return carry

    lax.fori_loop(0, nch, body, 0)


def stage_c_body(G_ref, table_ref, wphi_ref, Wg1T_ref, Wg2T_ref, out_ref):
    WgT = jnp.dot(Wg1T_ref[...], Wg2T_ref[...],
                  preferred_element_type=jnp.float32)
    tcn = table_ref[0][:, 0:64]
    wphin = wphi_ref[0]
    G = G_ref[0]

    def unpack(words):
        wi = lax.bitcast_convert_type(words, jnp.int32)
        lo = lax.bitcast_convert_type(wi << 16, jnp.float32)
        hi = lax.bitcast_convert_type(
            wi & jnp.int32(-65536), jnp.float32)
        return jnp.concatenate([lo, hi], axis=-1)

    tck = G[:, :, 0:64]
    wpsik = unpack(G[:, :, 64:96])
    alphak = unpack(G[:, :, 96:128])
    tcr = jnp.broadcast_to(tcn[:, None, :], (TQ2, KNN, 64))
    wpr = jnp.broadcast_to(wphin[:, None, :], (TQ2, KNN, 64))
    delta = jnp.maximum(tcr - tck, 0.0)
    gam = jnp.dot(delta.reshape(TQ2 * KNN, 64), WgT,
                  preferred_element_type=jnp.float32).reshape(TQ2, KNN, 64)
    gam = jnp.maximum(wpr - wpsik.astype(jnp.float32) + gam, 0.0)
    m = jnp.max(gam, axis=1, keepdims=True)
    e = jnp.exp(gam - m)
    ssum = jnp.sum(e, axis=1)
    acc = jnp.sum(e * (alphak.astype(jnp.float32) + delta), axis=1)
    out_ref[0] = acc / ssum


def kernel(features, coords, W_in, Wt1, Wt2, Wg1, Wg2):
    B, C_in, N = features.shape
    C_out = Wt1.shape[0]
    featsT = jnp.transpose(features, (0, 2, 1))
    coordsq = jnp.transpose(coords, (0, 2, 1))
    coordsq = jnp.pad(coordsq, ((0, 0), (0, 0), (0, 5)))
    coords8 = jnp.pad(coords, ((0, 0), (0, 5), (0, 0)))
    WinT = W_in.T
    Wt1T = Wt1.T
    Wt2T = Wt2.T
    Wg1T = Wg1.T
    Wg2T = Wg2.T

    grid_a = (B, N // TQ)
    table, wphi, idx = pl.pallas_call(
        stage_a_body,
        grid=grid_a,
        in_specs=[
            pl.BlockSpec((1, 8, N), lambda b, qi: (b, 0, 0)),
            pl.BlockSpec((1, TQ, 8), lambda b, qi: (b, qi, 0)),
            pl.BlockSpec((1, TQ, C_in), lambda b, qi: (b, qi, 0)),
            pl.BlockSpec((C_in, 3 * C_out), lambda b, qi: (0, 0)),
            pl.BlockSpec((3, C_out), lambda b, qi: (0, 0)),
            pl.BlockSpec((C_out, C_out), lambda b, qi: (0, 0)),
            pl.BlockSpec((C_out, C_out), lambda b, qi: (0, 0)),
            pl.BlockSpec((C_out, C_out), lambda b, qi: (0, 0)),
        ],
        out_specs=[
            pl.BlockSpec((1, TQ, 128), lambda b, qi: (b, qi, 0)),
            pl.BlockSpec((1, TQ, C_out), lambda b, qi: (b, qi, 0)),
            pl.BlockSpec((1, TQ, KNN), lambda b, qi: (b, qi, 0)),
        ],
        out_shape=[
            jax.ShapeDtypeStruct((B, N, 128), jnp.float32),
            jax.ShapeDtypeStruct((B, N, C_out), jnp.float32),
            jax.ShapeDtypeStruct((B, N, KNN), jnp.int32),
        ],
    )(coords8, coordsq, featsT, WinT, Wt1T, Wt2T, Wg1T, Wg2T)

    R = B * N * KNN
    tablep = table.reshape(B * N, 128)
    gather = pl.kernel(
        sc_gather_body,
        out_type=jax.ShapeDtypeStruct((R, 128), jnp.float32),
        mesh=plsc.VectorSubcoreMesh(core_axis_name="c", subcore_axis_name="s"),
        scratch_types=[
            pltpu.VMEM((SC_CHUNK,), jnp.int32),
            pltpu.VMEM((SC_CHUNK, 128), jnp.float32),
            pltpu.SemaphoreType.DMA,
        ],
    )
    G = gather(tablep, idx.reshape(R))

    grid_c = (B, N // TQ2)
    outT = pl.pallas_call(
        stage_c_body,
        grid=grid_c,
        in_specs=[
            pl.BlockSpec((1, TQ2, KNN, 128), lambda b, qi: (b, qi, 0, 0),),
            pl.BlockSpec((1, TQ2, 128), lambda b, qi: (b, qi, 0)),
            pl.BlockSpec((1, TQ2, C_out), lambda b, qi: (b, qi, 0)),
            pl.BlockSpec((C_out, C_out), lambda b, qi: (0, 0)),
            pl.BlockSpec((C_out, C_out), lambda b, qi: (0, 0)),
        ],
        out_specs=pl.BlockSpec((1, TQ2, C_out), lambda b, qi: (b, qi, 0)),
        out_shape=jax.ShapeDtypeStruct((B, N, C_out), jnp.float32),
    )(G.reshape(B, N, KNN, 128), table, wphi, Wg1T, Wg2T)

    return jnp.transpose(outT, (0, 2, 1))


# speedup vs baseline: 1.1660x; 1.1138x over previous
import functools

import jax
import jax.numpy as jnp
from jax import lax
from jax.experimental import pallas as pl
from jax.experimental.pallas import tpu as pltpu
from jax.experimental.pallas import tpu_sc as plsc

KNN = 16
TQ = 256
TQ2 = 256
SC_CHUNK = 128
TW = 256


def stage_a_body(coords_ref, coordsq_ref, featsT_ref, WinT_ref, Wt1T_ref,
                 Wt2T_ref, Wg1T_ref, Wg2T_ref, table_ref, wphi_ref, idx_ref):
    b = pl.program_id(0)
    n_all = coords_ref.shape[-1]
    fT = featsT_ref[0]
    linT = jnp.dot(fT, WinT_ref[...], preferred_element_type=jnp.float32)
    phiT = linT[:, 0:64]
    psiT = linT[:, 64:128]
    alphaT = linT[:, 128:192]
    WgT = jnp.dot(Wg1T_ref[...], Wg2T_ref[...],
                  preferred_element_type=jnp.float32)
    wphiT = jnp.dot(phiT, WgT, preferred_element_type=jnp.float32)
    wpsiT = jnp.dot(psiT, WgT, preferred_element_type=jnp.float32)
    q = coordsq_ref[0]
    t1 = (q[:, 0:1] * Wt1T_ref[0:1, :] + q[:, 1:2] * Wt1T_ref[1:2, :]
          + q[:, 2:3] * Wt1T_ref[2:3, :])
    tcT = jnp.dot(t1, Wt2T_ref[...], preferred_element_type=jnp.float32)

    def pack(cols):
        r = cols.astype(jnp.bfloat16).astype(jnp.float32)
        lo = lax.shift_right_logical(
            lax.bitcast_convert_type(r[:, 0:32], jnp.int32), 16)
        hi = lax.bitcast_convert_type(r[:, 32:64], jnp.int32)
        return lax.bitcast_convert_type(hi | lo, jnp.float32)

    table_ref[0] = jnp.concatenate([tcT, pack(wpsiT), pack(alphaT)], axis=1)
    wphi_ref[0] = wphiT
    a8 = coords_ref[0]
    inner = jnp.dot(q.astype(jnp.bfloat16), a8.astype(jnp.bfloat16),
                    preferred_element_type=jnp.float32)
    q2 = q[:, 0:1] * q[:, 0:1] + q[:, 1:2] * q[:, 1:2] + q[:, 2:3] * q[:, 2:3]
    a2 = (a8[0:1, :] * a8[0:1, :] + a8[1:2, :] * a8[1:2, :]
          + a8[2:3, :] * a8[2:3, :])
    d = (q2 - 2.0 * inner) + a2
    iota = lax.broadcasted_iota(jnp.int32, d.shape, 1)
    inf = jnp.float32(jnp.inf)
    cols = []
    for _ in range(KNN):
        ij = jnp.argmin(d, axis=1, keepdims=True).astype(jnp.int32)
        cols.append(ij)
        d = jnp.where(iota == ij, inf, d)
    idxs = jnp.concatenate(cols, axis=1)
    idx_ref[0] = idxs + b * n_all


def sc_gather_body(table_hbm, idx_hbm, out_hbm, idx_v, buf, sem):
    c = lax.axis_index("c")
    s = lax.axis_index("s")
    wid = s * 2 + c
    rows_total = idx_hbm.shape[0]
    rw = rows_total // 32
    nch = rw // SC_CHUNK

    def body(i, carry):
        off = wid * rw + i * SC_CHUNK
        pltpu.sync_copy(idx_hbm.at[pl.ds(off, SC_CHUNK)], idx_v)
        pltpu.async_copy(table_hbm.at[idx_v], buf, sem).wait()
        pltpu.sync_copy(buf, out_hbm.at[pl.ds(off, SC_CHUNK)])
        return carry

    lax.fori_loop(0, nch, body, 0)


def stage_c_body(G_ref, table_ref, wphi_ref, Wg1T_ref, Wg2T_ref, out_ref):
    WgT = jnp.dot(Wg1T_ref[...], Wg2T_ref[...],
                  preferred_element_type=jnp.float32)
    tcn = table_ref[0][:, 0:64]
    wphin = wphi_ref[0]
    G = G_ref[0]

    def unpack(words):
        wi = lax.bitcast_convert_type(words, jnp.int32)
        lo = lax.bitcast_convert_type(wi << 16, jnp.float32)
        hi = lax.bitcast_convert_type(
            wi & jnp.int32(-65536), jnp.float32)
        return jnp.concatenate([lo, hi], axis=-1)

    tck = G[:, :, 0:64]
    wpsik = unpack(G[:, :, 64:96])
    alphak = unpack(G[:, :, 96:128])
    tcr = jnp.broadcast_to(tcn[:, None, :], (TQ2, KNN, 64))
    wpr = jnp.broadcast_to(wphin[:, None, :], (TQ2, KNN, 64))
    delta = jnp.maximum(tcr - tck, 0.0)
    gam = jnp.dot(delta.reshape(TQ2 * KNN, 64), WgT,
                  preferred_element_type=jnp.float32).reshape(TQ2, KNN, 64)
    gam = jnp.maximum(wpr - wpsik.astype(jnp.float32) + gam, 0.0)
    m = jnp.max(gam, axis=1, keepdims=True)
    e = jnp.exp(gam - m)
    ssum = jnp.sum(e, axis=1)
    acc = jnp.sum(e * (alphak.astype(jnp.float32) + delta), axis=1)
    out_ref[0] = acc / ssum


def kernel(features, coords, W_in, Wt1, Wt2, Wg1, Wg2):
    B, C_in, N = features.shape
    C_out = Wt1.shape[0]
    featsT = jnp.transpose(features, (0, 2, 1))
    coordsq = jnp.transpose(coords, (0, 2, 1))
    coordsq = jnp.pad(coordsq, ((0, 0), (0, 0), (0, 5)))
    coords8 = jnp.pad(coords, ((0, 0), (0, 5), (0, 0)))
    WinT = W_in.T
    Wt1T = Wt1.T
    Wt2T = Wt2.T
    Wg1T = Wg1.T
    Wg2T = Wg2.T

    stage_a = pl.pallas_call(
        stage_a_body,
        grid=(1, N // TQ),
        in_specs=[
            pl.BlockSpec((1, 8, N), lambda b, qi: (b, 0, 0)),
            pl.BlockSpec((1, TQ, 8), lambda b, qi: (b, qi, 0)),
            pl.BlockSpec((1, TQ, C_in), lambda b, qi: (b, qi, 0)),
            pl.BlockSpec((C_in, 3 * C_out), lambda b, qi: (0, 0)),
            pl.BlockSpec((3, C_out), lambda b, qi: (0, 0)),
            pl.BlockSpec((C_out, C_out), lambda b, qi: (0, 0)),
            pl.BlockSpec((C_out, C_out), lambda b, qi: (0, 0)),
            pl.BlockSpec((C_out, C_out), lambda b, qi: (0, 0)),
        ],
        out_specs=[
            pl.BlockSpec((1, TQ, 128), lambda b, qi: (b, qi, 0)),
            pl.BlockSpec((1, TQ, C_out), lambda b, qi: (b, qi, 0)),
            pl.BlockSpec((1, TQ, KNN), lambda b, qi: (b, qi, 0)),
        ],
        out_shape=[
            jax.ShapeDtypeStruct((1, N, 128), jnp.float32),
            jax.ShapeDtypeStruct((1, N, C_out), jnp.float32),
            jax.ShapeDtypeStruct((1, N, KNN), jnp.int32),
        ],
    )

    Rb = N * KNN
    gather = pl.kernel(
        sc_gather_body,
        out_type=jax.ShapeDtypeStruct((Rb, 128), jnp.float32),
        mesh=plsc.VectorSubcoreMesh(core_axis_name="c", subcore_axis_name="s"),
        scratch_types=[
            pltpu.VMEM((SC_CHUNK,), jnp.int32),
            pltpu.VMEM((SC_CHUNK, 128), jnp.float32),
            pltpu.SemaphoreType.DMA,
        ],
    )

    stage_c = pl.pallas_call(
        stage_c_body,
        grid=(1, N // TQ2),
        in_specs=[
            pl.BlockSpec((1, TQ2, KNN, 128), lambda b, qi: (b, qi, 0, 0),),
            pl.BlockSpec((1, TQ2, 128), lambda b, qi: (b, qi, 0)),
            pl.BlockSpec((1, TQ2, C_out), lambda b, qi: (b, qi, 0)),
            pl.BlockSpec((C_out, C_out), lambda b, qi: (0, 0)),
            pl.BlockSpec((C_out, C_out), lambda b, qi: (0, 0)),
        ],
        out_specs=pl.BlockSpec((1, TQ2, C_out), lambda b, qi: (b, qi, 0)),
        out_shape=jax.ShapeDtypeStruct((1, N, C_out), jnp.float32),
    )

    outs = []
    for b in range(B):
        table, wphi, idx = stage_a(
            coords8[b:b + 1], coordsq[b:b + 1], featsT[b:b + 1],
            WinT, Wt1T, Wt2T, Wg1T, Wg2T)
        G = gather(table.reshape(N, 128), idx.reshape(Rb))
        outs.append(stage_c(
            G.reshape(1, N, KNN, 128), table, wphi, Wg1T, Wg2T))
    return jnp.transpose(jnp.concatenate(outs, axis=0), (0, 2, 1))
